# baseline (speedup 1.0000x reference)
import functools

import jax
import jax.numpy as jnp
from jax import lax
from jax.experimental import pallas as pl
from jax.experimental.pallas import tpu as pltpu
from jax.experimental.pallas import tpu_sc as plsc

BATCH = 16384
DIM = 64
LANES = 16
CHUNK = 128

_info = plsc.get_sparse_core_info()
_NC, _NS = _info.num_cores, _info.num_subcores
_NW = _NC * _NS
_BPW = BATCH // _NW
_NCHUNK = _BPW // CHUNK
_NGROUP = _BPW // LANES


def _gmf_body(user2, item2, ut, it, w64, b16, out,
              uidx_v, iidx_v, urows, irows, w_v, b_v, out_v, tr_v,
              sem_u, sem_i):
    wid = lax.axis_index("s") * _NC + lax.axis_index("c")

    pltpu.sync_copy(user2.at[pl.ds(wid * _NCHUNK, _NCHUNK)], uidx_v)
    pltpu.sync_copy(item2.at[pl.ds(wid * _NCHUNK, _NCHUNK)], iidx_v)
    pltpu.sync_copy(w64, w_v)
    pltpu.sync_copy(b16, b_v)

    copies = []
    for c in range(_NCHUNK):
        copies.append(pltpu.async_copy(
            ut.at[uidx_v.at[c]], urows.at[pl.ds(c * CHUNK, CHUNK)], sem_u))
        copies.append(pltpu.async_copy(
            it.at[iidx_v.at[c]], irows.at[pl.ds(c * CHUNK, CHUNK)], sem_i))
    for cp in copies:
        cp.wait()

    wvs = [w_v[pl.ds(j * LANES, LANES)] for j in range(DIM // LANES)]
    bv = b_v[...]
    scat_idx = lax.iota(jnp.int32, LANES) * LANES

    def group(g, carry):
        for b_local in range(LANES):
            b = g * LANES + b_local
            acc = jnp.zeros((LANES,), jnp.float32)
            for j in range(DIM // LANES):
                uv = urows[b, pl.ds(j * LANES, LANES)]
                iv = irows[b, pl.ds(j * LANES, LANES)]
                acc = acc + uv * iv * wvs[j]
            plsc.store_scatter(tr_v, [scat_idx + b_local], acc)
        tot = tr_v[pl.ds(0, LANES)]
        for l in range(1, LANES):
            tot = tot + tr_v[pl.ds(l * LANES, LANES)]
        x = tot + bv
        out_v[pl.ds(g * LANES, LANES)] = 1.0 / (1.0 + jnp.exp(-x))
        return carry

    lax.fori_loop(0, _NGROUP, group, 0)

    pltpu.sync_copy(out_v, out.at[pl.ds(wid * _BPW, _BPW)])


@jax.jit
def _gmf_sc(user2, item2, user_table, item_table, w64, b16):
    mesh = plsc.VectorSubcoreMesh(core_axis_name="c", subcore_axis_name="s")
    run = functools.partial(
        pl.kernel,
        mesh=mesh,
        out_type=jax.ShapeDtypeStruct((BATCH,), jnp.float32),
        scratch_types=[
            pltpu.VMEM((_NCHUNK, CHUNK), jnp.int32),
            pltpu.VMEM((_NCHUNK, CHUNK), jnp.int32),
            pltpu.VMEM((_BPW, DIM), jnp.float32),
            pltpu.VMEM((_BPW, DIM), jnp.float32),
            pltpu.VMEM((DIM,), jnp.float32),
            pltpu.VMEM((LANES,), jnp.float32),
            pltpu.VMEM((_BPW,), jnp.float32),
            pltpu.VMEM((LANES * LANES,), jnp.float32),
            pltpu.SemaphoreType.DMA,
            pltpu.SemaphoreType.DMA,
        ],
        compiler_params=pltpu.CompilerParams(
            needs_layout_passes=False, use_tc_tiling_on_sc=False),
    )(_gmf_body)
    return run(user2, item2, user_table, item_table, w64, b16)


def kernel(user, item, user_table, item_table, dense_w, dense_b):
    user2 = user.astype(jnp.int32).reshape(_NW * _NCHUNK, CHUNK)
    item2 = item.astype(jnp.int32).reshape(_NW * _NCHUNK, CHUNK)
    w64 = dense_w.reshape(DIM)
    b16 = jnp.broadcast_to(dense_b, (LANES,))
    return _gmf_sc(user2, item2, user_table, item_table, w64, b16)


# speedup vs baseline: 1.5649x; 1.5649x over previous
import functools

import jax
import jax.numpy as jnp
from jax import lax
from jax.experimental import pallas as pl
from jax.experimental.pallas import tpu as pltpu
from jax.experimental.pallas import tpu_sc as plsc

BATCH = 16384
DIM = 64
LANES = 16

_info = plsc.get_sparse_core_info()
_NC, _NS = _info.num_cores, _info.num_subcores
_NW = _NC * _NS
_BPW = BATCH // _NW
_NCHUNK = 2
_CB = _BPW // _NCHUNK
_NGROUP = _CB // LANES


def _gmf_body(user_h, item_h, ut, it, w64, b16, out,
              uidx_v, iidx_v, urows, irows, w_v, b_v, out_v, tr_v,
              sem_u, sem_i):
    wid = lax.axis_index("s") * _NC + lax.axis_index("c")
    base = wid * _BPW

    pltpu.sync_copy(user_h.at[pl.ds(base, _BPW)], uidx_v)
    pltpu.sync_copy(item_h.at[pl.ds(base, _BPW)], iidx_v)
    pltpu.sync_copy(w64, w_v)
    pltpu.sync_copy(b16, b_v)

    wvs = [w_v[pl.ds(j * LANES, LANES)] for j in range(DIM // LANES)]
    bv = b_v[...]
    scat_idx = lax.iota(jnp.int32, LANES) * LANES

    for half in range(_NCHUNK):
        hbase = half * _CB

        def issue(g, carry, hbase=hbase):
            uvec = uidx_v[pl.ds(hbase + g * LANES, LANES)]
            ivec = iidx_v[pl.ds(hbase + g * LANES, LANES)]
            for l in range(LANES):
                b = g * LANES + l
                pltpu.async_copy(ut.at[uvec[l]], urows.at[b], sem_u)
                pltpu.async_copy(it.at[ivec[l]], irows.at[b], sem_i)
            return carry

        lax.fori_loop(0, _NGROUP, issue, 0)
        pltpu.make_async_copy(ut.at[pl.ds(0, _CB)], urows, sem_u).wait()
        pltpu.make_async_copy(it.at[pl.ds(0, _CB)], irows, sem_i).wait()

        def group(g, carry, hbase=hbase):
            for b_local in range(LANES):
                b = g * LANES + b_local
                acc = jnp.zeros((LANES,), jnp.float32)
                for j in range(DIM // LANES):
                    uv = urows[b, pl.ds(j * LANES, LANES)]
                    iv = irows[b, pl.ds(j * LANES, LANES)]
                    acc = acc + uv * iv * wvs[j]
                plsc.store_scatter(tr_v, [scat_idx + b_local], acc)
            tot = tr_v[pl.ds(0, LANES)]
            for l in range(1, LANES):
                tot = tot + tr_v[pl.ds(l * LANES, LANES)]
            x = tot + bv
            out_v[pl.ds(hbase + g * LANES, LANES)] = 1.0 / (1.0 + jnp.exp(-x))
            return carry

        lax.fori_loop(0, _NGROUP, group, 0)

    pltpu.sync_copy(out_v, out.at[pl.ds(base, _BPW)])


@jax.jit
def _gmf_sc(user, item, user_table, item_table, w64, b16):
    mesh = plsc.VectorSubcoreMesh(core_axis_name="c", subcore_axis_name="s")
    run = functools.partial(
        pl.kernel,
        mesh=mesh,
        out_type=jax.ShapeDtypeStruct((BATCH,), jnp.float32),
        scratch_types=[
            pltpu.VMEM((_BPW,), jnp.int32),
            pltpu.VMEM((_BPW,), jnp.int32),
            pltpu.VMEM((_CB, DIM), jnp.float32),
            pltpu.VMEM((_CB, DIM), jnp.float32),
            pltpu.VMEM((DIM,), jnp.float32),
            pltpu.VMEM((LANES,), jnp.float32),
            pltpu.VMEM((_BPW,), jnp.float32),
            pltpu.VMEM((LANES * LANES,), jnp.float32),
            pltpu.SemaphoreType.DMA,
            pltpu.SemaphoreType.DMA,
        ],
        compiler_params=pltpu.CompilerParams(needs_layout_passes=False),
    )(_gmf_body)
    return run(user, item, user_table, item_table, w64, b16)


def kernel(user, item, user_table, item_table, dense_w, dense_b):
    w64 = dense_w.reshape(DIM)
    b16 = jnp.broadcast_to(dense_b, (LANES,))
    return _gmf_sc(user.astype(jnp.int32), item.astype(jnp.int32),
                   user_table, item_table, w64, b16)
